# Initial kernel scaffold; baseline (speedup 1.0000x reference)
import functools

import jax
import jax.numpy as jnp
from jax import lax
from jax.experimental import pallas as pl
from jax.experimental.pallas import tpu as pltpu
from jax.experimental.pallas import tpu_sc as plsc

B, L, F = 1024, 20, 26
NUM_FEAT, VOCAB, E = 128, 100000, 32
N = B * L

NC, NS = 2, 16
NW = NC * NS
N_PER_W = N // NW
GP = 4
G_IDX = GP * F
NG = N_PER_W // GP


def _mm_body(x_ref, w_ref, o_ref):
    o_ref[...] = lax.dot_general(
        x_ref[...], w_ref[...],
        (((1,), (1,)), ((), ())),
        preferred_element_type=jnp.float32,
    )


def _matmul(x, w):
    blk = 2048
    return pl.pallas_call(
        _mm_body,
        grid=(N // blk,),
        in_specs=[
            pl.BlockSpec((blk, NUM_FEAT), lambda i: (i, 0)),
            pl.BlockSpec((E, NUM_FEAT), lambda i: (0, 0)),
        ],
        out_specs=pl.BlockSpec((blk, E), lambda i: (i, 0)),
        out_shape=jax.ShapeDtypeStruct((N, E), jnp.float32),
    )(x, w)


def _sc_body(table_hbm, idx_hbm, e2_hbm, out_hbm, idx_v, acc_v, rows0, rows1,
             sem0, sem1):
    cid = lax.axis_index("core")
    sid = lax.axis_index("subcore")
    wid = sid * NC + cid
    base = wid * (N_PER_W * E)

    pltpu.sync_copy(idx_hbm.at[wid], idx_v)
    pltpu.sync_copy(e2_hbm.at[pl.ds(base, N_PER_W * E)], acc_v)

    def fire(g, buf, sem):
        pltpu.async_copy(table_hbm.at[idx_v.at[g]], buf, sem)

    def wait(g, buf, sem):
        pltpu.make_async_copy(table_hbm.at[idx_v.at[g]], buf, sem).wait()

    def accum(g, buf):
        for q in range(GP):
            off = (g * GP + q) * E
            a0 = acc_v[pl.ds(off, 16)]
            a1 = acc_v[pl.ds(off + 16, 16)]
            for f in range(F):
                r = q * F + f
                a0 = a0 + buf[r, pl.ds(0, 16)]
                a1 = a1 + buf[r, pl.ds(16, 16)]
            acc_v[pl.ds(off, 16)] = a0
            acc_v[pl.ds(off + 16, 16)] = a1

    fire(0, rows0, sem0)
    fire(1, rows1, sem1)

    @pl.loop(0, NG, step=2)
    def _(g):
        wait(g, rows0, sem0)
        accum(g, rows0)

        @pl.when(g + 2 < NG)
        def _():
            fire(g + 2, rows0, sem0)

        wait(g + 1, rows1, sem1)
        accum(g + 1, rows1)

        @pl.when(g + 3 < NG)
        def _():
            fire(g + 3, rows1, sem1)

    pltpu.sync_copy(acc_v, out_hbm.at[pl.ds(base, N_PER_W * E)])


@jax.jit
def kernel(nodes_numerical, nodes_categorical, W_num, tables):
    x = nodes_numerical.reshape(N, NUM_FEAT)
    e2 = _matmul(x, W_num).reshape(N * E)

    idx_g = nodes_categorical.reshape(N, F) + (
        jnp.arange(F, dtype=jnp.int32) * VOCAB)[None, :]
    idx_arr = idx_g.reshape(NW, NG, G_IDX)
    table_flat = tables.reshape(F * VOCAB, E)

    mesh = plsc.VectorSubcoreMesh(core_axis_name="core",
                                  subcore_axis_name="subcore")
    sc = pl.kernel(
        _sc_body,
        out_type=jax.ShapeDtypeStruct((N * E,), jnp.float32),
        mesh=mesh,
        scratch_types=[
            pltpu.VMEM((NG, G_IDX), jnp.int32),
            pltpu.VMEM((N_PER_W * E,), jnp.float32),
            pltpu.VMEM((G_IDX, E), jnp.float32),
            pltpu.VMEM((G_IDX, E), jnp.float32),
            pltpu.SemaphoreType.DMA,
            pltpu.SemaphoreType.DMA,
        ],
    )
    out = sc(table_flat, idx_arr, e2)
    return out.reshape(B, L, E)


# speedup vs baseline: 1.2328x; 1.2328x over previous
import functools

import jax
import jax.numpy as jnp
from jax import lax
from jax.experimental import pallas as pl
from jax.experimental.pallas import tpu as pltpu
from jax.experimental.pallas import tpu_sc as plsc

B, L, F = 1024, 20, 26
NUM_FEAT, VOCAB, E = 128, 100000, 32
N = B * L

NC, NS = 2, 16
NW = NC * NS
N_PER_W = N // NW
GP = 4
G_IDX = GP * F
NG = N_PER_W // GP


def _mm_body(x_ref, w_ref, o_ref):
    o_ref[...] = lax.dot_general(
        x_ref[...], w_ref[...],
        (((1,), (1,)), ((), ())),
        preferred_element_type=jnp.float32,
    )


def _matmul(x, w):
    blk = 2048
    return pl.pallas_call(
        _mm_body,
        grid=(N // blk,),
        in_specs=[
            pl.BlockSpec((blk, NUM_FEAT), lambda i: (i, 0)),
            pl.BlockSpec((E, NUM_FEAT), lambda i: (0, 0)),
        ],
        out_specs=pl.BlockSpec((blk, E), lambda i: (i, 0)),
        out_shape=jax.ShapeDtypeStruct((N, E), jnp.float32),
    )(x, w)


def _sc_body(table_hbm, idx_hbm, e2_hbm, out_hbm, idx_v, acc_v, rows0, rows1,
             sem0, sem1):
    cid = lax.axis_index("core")
    sid = lax.axis_index("subcore")
    wid = sid * NC + cid
    base = wid * (N_PER_W * E)

    pltpu.sync_copy(idx_hbm.at[wid], idx_v)
    pltpu.sync_copy(e2_hbm.at[pl.ds(base, N_PER_W * E)], acc_v)

    def fire(g, buf, sem):
        pltpu.async_copy(table_hbm.at[idx_v.at[g]], buf, sem)

    def wait(g, buf, sem):
        pltpu.make_async_copy(table_hbm.at[idx_v.at[g]], buf, sem).wait()

    def accum(g, buf):
        for q in range(GP):
            off = (g * GP + q) * E
            a0 = acc_v[pl.ds(off, 16)]
            a1 = acc_v[pl.ds(off + 16, 16)]
            for f in range(F):
                r = q * F + f
                a0 = a0 + buf[r, pl.ds(0, 16)]
                a1 = a1 + buf[r, pl.ds(16, 16)]
            acc_v[pl.ds(off, 16)] = a0
            acc_v[pl.ds(off + 16, 16)] = a1

    fire(0, rows0, sem0)
    fire(1, rows1, sem1)

    @pl.loop(0, NG, step=2)
    def _(g):
        wait(g, rows0, sem0)
        accum(g, rows0)

        @pl.when(g + 2 < NG)
        def _():
            fire(g + 2, rows0, sem0)

        wait(g + 1, rows1, sem1)
        accum(g + 1, rows1)

        @pl.when(g + 3 < NG)
        def _():
            fire(g + 3, rows1, sem1)

    pltpu.sync_copy(acc_v, out_hbm.at[pl.ds(base, N_PER_W * E)])


@jax.jit
def kernel(nodes_numerical, nodes_categorical, W_num, tables):
    x = nodes_numerical.reshape(N, NUM_FEAT)
    e2 = _matmul(x, W_num).reshape(N * E)

    idx_g = nodes_categorical.reshape(N, F) + (
        jnp.arange(F, dtype=jnp.int32) * VOCAB)[None, :]
    idx_arr = idx_g.reshape(NW, NG, G_IDX)
    table_flat = tables.reshape(F * VOCAB, E)

    mesh = plsc.VectorSubcoreMesh(core_axis_name="core",
                                  subcore_axis_name="subcore")
    sc = pl.kernel(
        _sc_body,
        out_type=jax.ShapeDtypeStruct((N * E,), jnp.float32),
        mesh=mesh,
        scratch_types=[
            pltpu.VMEM((NG, G_IDX), jnp.int32),
            pltpu.VMEM((N_PER_W * E,), jnp.float32),
            pltpu.VMEM((G_IDX, E), jnp.float32),
            pltpu.VMEM((G_IDX, E), jnp.float32),
            pltpu.SemaphoreType.DMA,
            pltpu.SemaphoreType.DMA,
        ],
        compiler_params=pltpu.CompilerParams(use_tc_tiling_on_sc=False),
    )
    out = sc(table_flat, idx_arr, e2)
    return out.reshape(B, L, E)
